# Initial kernel scaffold; baseline (speedup 1.0000x reference)
import jax
import jax.numpy as jnp
from jax.experimental import pallas as pl


def kernel(inputs, h_tm1, c_tm1, r_tm1, kernel, recurrent_kernel, bias, write_gate, memory, wu):
    raise NotImplementedError("write your pallas kernel here")


# speedup vs baseline: 1.5614x; 1.5614x over previous
import functools

import jax
import jax.numpy as jnp
from jax.experimental import pallas as pl
from jax.experimental.pallas import tpu as pltpu


def _hard_sigmoid(x):
    return jnp.clip(0.2 * x + 0.5, 0.0, 1.0)


def _mann_body(Ts, T, b, u,
               inputs_ref, h_tm1_ref, c_tm1_ref, r_tm1_ref, w_ref, rk_ref,
               b_ref, wg_ref, mem_ref, wu_ref,
               h_out, c_out, r_out, memnew_out, wunew_out,
               mem_copy, sim_s, wu_copy, keyn_s, m_s, l_s, minv_s, mini_s):
    phase = pl.program_id(0)
    t = pl.program_id(1)

    @pl.when((phase == 0) & (t == 0))
    def _lstm():
        x = jnp.dot(inputs_ref[...], w_ref[...],
                    preferred_element_type=jnp.float32) + b_ref[...]
        rk = rk_ref[...]
        hr = jnp.dot(h_tm1_ref[...], rk[:, :4 * u],
                     preferred_element_type=jnp.float32)
        rr = jnp.dot(r_tm1_ref[...], rk[:, 4 * u:],
                     preferred_element_type=jnp.float32)
        i = _hard_sigmoid(x[:, :u] + hr[:, :u] + rr)
        f = _hard_sigmoid(x[:, u:2 * u] + hr[:, u:2 * u])
        c = f * c_tm1_ref[...] + i * jnp.tanh(x[:, 2 * u:3 * u] + hr[:, 2 * u:3 * u])
        o = _hard_sigmoid(x[:, 3 * u:] + hr[:, 3 * u:])
        h = o * jnp.tanh(c)
        h_out[...] = h
        c_out[...] = c
        nrm = jnp.sqrt(jnp.sum(h * h, axis=1, keepdims=True))
        keyn_s[...] = h / (nrm + 1e-8)
        m_s[...] = jnp.full((b, 128), -jnp.inf, jnp.float32)
        l_s[...] = jnp.zeros((b, 128), jnp.float32)
        minv_s[...] = jnp.full((b, 128), jnp.inf, jnp.float32)
        mini_s[...] = jnp.zeros((b, 128), jnp.int32)

    @pl.when(phase == 0)
    def _p0():
        mem_t = mem_ref[...]
        mem_copy[pl.ds(t * Ts, Ts), :] = mem_t
        nrm = jnp.sqrt(jnp.sum(mem_t * mem_t, axis=1, keepdims=True)) + 1e-8
        mem_n = mem_t / nrm
        s = jax.lax.dot_general(keyn_s[...], mem_n, (((1,), (1,)), ((), ())),
                                preferred_element_type=jnp.float32)
        sim_s[:, pl.ds(t * Ts, Ts)] = s
        m_old = m_s[:, 0:1]
        l_old = l_s[:, 0:1]
        m_new = jnp.maximum(m_old, jnp.max(s, axis=1, keepdims=True))
        l_new = (l_old * jnp.exp(m_old - m_new)
                 + jnp.sum(jnp.exp(s - m_new), axis=1, keepdims=True))
        m_s[...] = jnp.broadcast_to(m_new, (b, 128))
        l_s[...] = jnp.broadcast_to(l_new, (b, 128))
        wu_t = wu_ref[...]
        wu_copy[:, pl.ds(t * Ts, Ts)] = wu_t
        tmin = jnp.min(wu_t, axis=1, keepdims=True)
        lanes = jax.lax.broadcasted_iota(jnp.int32, (b, Ts), 1)
        tidx = jnp.min(jnp.where(wu_t == tmin, lanes, jnp.int32(2 ** 30)),
                       axis=1, keepdims=True) + t * Ts
        better = tmin < minv_s[:, 0:1]
        mini_s[...] = jnp.broadcast_to(
            jnp.where(better, tidx, mini_s[:, 0:1]), (b, 128))
        minv_s[...] = jnp.broadcast_to(
            jnp.where(better, tmin, minv_s[:, 0:1]), (b, 128))

    @pl.when(phase == 1)
    def _p1():
        mem_t = mem_copy[pl.ds(t * Ts, Ts), :]
        s = sim_s[:, pl.ds(t * Ts, Ts)]
        wr = jnp.exp(s - m_s[:, 0:1]) * (1.0 / l_s[:, 0:1])
        rc = jnp.dot(wr, mem_t, preferred_element_type=jnp.float32)

        @pl.when(t == 0)
        def _():
            r_out[...] = rc

        @pl.when(t != 0)
        def _():
            r_out[...] = r_out[...] + rc

        lanes = jax.lax.broadcasted_iota(jnp.int32, (b, Ts), 1) + t * Ts
        wlu = (lanes == mini_s[:, 0:1]).astype(jnp.float32)
        sg = 1.0 / (1.0 + jnp.exp(-wg_ref[...]))
        ww = sg * wr + (1.0 - sg) * wlu
        lu_row = jnp.transpose(mini_s[...].astype(jnp.float32))[0:1, :]
        slot_col = (jax.lax.broadcasted_iota(jnp.int32, (Ts, b), 0)
                    .astype(jnp.float32) + t * Ts)
        hit = jnp.max((slot_col == lu_row).astype(jnp.float32),
                      axis=1, keepdims=True)
        mem_e = jnp.where(hit > 0.0, 0.0, mem_t)
        upd = jax.lax.dot_general(ww, h_out[...], (((0,), (0,)), ((), ())),
                                  preferred_element_type=jnp.float32)
        memnew_out[...] = mem_e + upd
        wunew_out[...] = 0.5 * wu_copy[:, pl.ds(t * Ts, Ts)] + wr + ww


def kernel(inputs, h_tm1, c_tm1, r_tm1, kernel, recurrent_kernel, bias,
           write_gate, memory, wu):
    n_slots, u = memory.shape
    b = inputs.shape[0]
    Ts = 2048 if n_slots % 2048 == 0 else n_slots
    T = n_slots // Ts
    bias2 = bias.reshape(1, 4 * u)
    wg2 = write_gate.reshape(1, 1)

    const = lambda p, t: (0, 0)
    outs = pl.pallas_call(
        functools.partial(_mann_body, Ts, T, b, u),
        grid=(2, T),
        in_specs=[
            pl.BlockSpec(inputs.shape, const),
            pl.BlockSpec(h_tm1.shape, const),
            pl.BlockSpec(c_tm1.shape, const),
            pl.BlockSpec(r_tm1.shape, const),
            pl.BlockSpec(kernel.shape, const),
            pl.BlockSpec(recurrent_kernel.shape, const),
            pl.BlockSpec((1, 4 * u), const),
            pl.BlockSpec((1, 1), const),
            pl.BlockSpec((Ts, u), lambda p, t: (jnp.where(p == 0, t, T - 1), 0)),
            pl.BlockSpec((b, Ts), lambda p, t: (0, jnp.where(p == 0, t, T - 1))),
        ],
        out_specs=[
            pl.BlockSpec((b, u), const),
            pl.BlockSpec((b, u), const),
            pl.BlockSpec((b, u), const),
            pl.BlockSpec((Ts, u), lambda p, t: (jnp.where(p == 0, 0, t), 0)),
            pl.BlockSpec((b, Ts), lambda p, t: (0, jnp.where(p == 0, 0, t))),
        ],
        out_shape=[
            jax.ShapeDtypeStruct((b, u), jnp.float32),
            jax.ShapeDtypeStruct((b, u), jnp.float32),
            jax.ShapeDtypeStruct((b, u), jnp.float32),
            jax.ShapeDtypeStruct((n_slots, u), jnp.float32),
            jax.ShapeDtypeStruct((b, n_slots), jnp.float32),
        ],
        scratch_shapes=[
            pltpu.VMEM((n_slots, u), jnp.float32),
            pltpu.VMEM((b, n_slots), jnp.float32),
            pltpu.VMEM((b, n_slots), jnp.float32),
            pltpu.VMEM((b, 128), jnp.float32),
            pltpu.VMEM((b, 128), jnp.float32),
            pltpu.VMEM((b, 128), jnp.float32),
            pltpu.VMEM((b, 128), jnp.float32),
            pltpu.VMEM((b, 128), jnp.int32),
        ],
    )(inputs, h_tm1, c_tm1, r_tm1, kernel, recurrent_kernel, bias2, wg2,
      memory, wu)
    h, c, r, mem_new, wu_new = outs
    return h, c, r, mem_new, wu_new
